# baseline (speedup 1.0000x reference)
import jax
import jax.numpy as jnp
from jax.experimental import pallas as pl
from jax.experimental.pallas import tpu as pltpu


def _make_body(n, ck, depth):
    def body(x_hbm, o_ref, x_buf, in_sem):
        def dma_in(j):
            s = j % depth
            pltpu.make_async_copy(
                x_hbm.at[pl.ds(j * ck, ck)], x_buf.at[s], in_sem.at[s]).start()

        def wait_in(j):
            s = j % depth
            pltpu.make_async_copy(
                x_hbm.at[pl.ds(j * ck, ck)], x_buf.at[s], in_sem.at[s]).wait()

        for j in range(depth):
            dma_in(j)
        for j in range(n):
            wait_in(j)
            if j + depth < n:
                dma_in(j + depth)
        o_ref[...] = x_buf[0, :1, :8, :]

    return body


def kernel(x, w1, b1, w2, b2):
    B, C, H, W = x.shape
    R = C * H * W // 128
    x2 = x.reshape(B, R, 128)
    ck = 16
    n = B // ck
    depth = 4
    out = pl.pallas_call(
        _make_body(n, ck, depth),
        out_shape=jax.ShapeDtypeStruct((1, 8, 128), x.dtype),
        in_specs=[pl.BlockSpec(memory_space=pl.ANY)],
        out_specs=pl.BlockSpec(memory_space=pltpu.MemorySpace.VMEM),
        scratch_shapes=[
            pltpu.VMEM((depth, ck, R, 128), x.dtype),
            pltpu.SemaphoreType.DMA((depth,)),
        ],
        compiler_params=pltpu.CompilerParams(
            vmem_limit_bytes=60 * 1024 * 1024,
        ),
    )(x2)
    return out


# speedup vs baseline: 1.3824x; 1.3824x over previous
import jax
import jax.numpy as jnp
from jax.experimental import pallas as pl
from jax.experimental.pallas import tpu as pltpu


def _se_body(x_ref, w1_ref, b1_ref, w2_ref, b2_ref, o_ref):
    x = x_ref[...]
    pooled = jnp.mean(x, axis=1)
    h = jnp.dot(pooled, w1_ref[...], preferred_element_type=jnp.float32)
    h = jnp.maximum(h + b1_ref[...], 0.0)
    g = jnp.dot(h, w2_ref[...], preferred_element_type=jnp.float32)
    g = jax.nn.sigmoid(g + b2_ref[...])
    o_ref[...] = (x * g[:, None, :].astype(x.dtype)).astype(o_ref.dtype)


def kernel(x, w1, b1, w2, b2):
    B, C, H, W = x.shape
    HW = H * W
    hidden = w1.shape[1]
    itemsize = jnp.dtype(x.dtype).itemsize

    xt = x.reshape(B, C, HW).transpose(0, 2, 1)

    lanes = -(-C // 128) * 128
    sub = -(-HW // 8) * 8
    win = sub * lanes * itemsize
    max_bt = (50 * 1024 * 1024) // (4 * win)
    bt = int(max(1, min(max_bt, pl.cdiv(B, 2))))
    grid = (int(pl.cdiv(B, bt)),)

    block = (bt, HW, C)
    out = pl.pallas_call(
        _se_body,
        out_shape=jax.ShapeDtypeStruct((B, HW, C), x.dtype),
        grid=grid,
        in_specs=[
            pl.BlockSpec(block, lambda b: (b, 0, 0)),
            pl.BlockSpec((C, hidden), lambda b: (0, 0)),
            pl.BlockSpec((1, hidden), lambda b: (0, 0)),
            pl.BlockSpec((hidden, C), lambda b: (0, 0)),
            pl.BlockSpec((1, C), lambda b: (0, 0)),
        ],
        out_specs=pl.BlockSpec(block, lambda b: (b, 0, 0)),
        compiler_params=pltpu.CompilerParams(
            dimension_semantics=("parallel",),
            vmem_limit_bytes=60 * 1024 * 1024,
        ),
        cost_estimate=pl.CostEstimate(
            flops=3 * B * C * HW + 4 * B * C * hidden,
            transcendentals=B * C,
            bytes_accessed=2 * B * C * HW * itemsize,
        ),
    )(xt, w1, b1.reshape(1, hidden), w2, b2.reshape(1, C))

    return out.transpose(0, 2, 1).reshape(B, C, H, W)
